# baseline (speedup 1.0000x reference)
import functools

import jax
import jax.numpy as jnp
from jax import lax
from jax.experimental import pallas as pl
from jax.experimental.pallas import tpu as pltpu
from jax.experimental.pallas import tpu_sc as plsc

BATCH = 16384
LATENT_DIM = 32
NC = 2
NS = 16
NW = NC * NS
B_PER_W = BATCH // NW
N_GROUPS = B_PER_W // 16


def _sigmoid(x):
    return 1.0 / (1.0 + jnp.exp(-x))


def _body(user_hbm, item_hbm, pro_hbm, diff_hbm, k_hbm, out_hbm,
          u_idx, it_idx, pro_v, diff_v, k_v, out_v, sem0, sem1, sem2):
    wid = lax.axis_index("s") * NC + lax.axis_index("c")
    base = wid * B_PER_W

    pltpu.sync_copy(user_hbm.at[pl.ds(base, B_PER_W)], u_idx)
    pltpu.sync_copy(item_hbm.at[pl.ds(base, B_PER_W)], it_idx)

    cp0 = pltpu.make_async_copy(pro_hbm.at[u_idx], pro_v, sem0)
    cp1 = pltpu.make_async_copy(diff_hbm.at[it_idx], diff_v, sem1)
    cp2 = pltpu.make_async_copy(k_hbm.at[it_idx], k_v, sem2)
    cp0.start()
    cp1.start()
    cp2.start()
    cp0.wait()
    cp1.wait()
    cp2.wait()

    lane = lax.iota(jnp.int32, 16)

    def group(g, carry):
        rows = g * 16 + lane
        accp = jnp.zeros((16,), jnp.float32)
        accd = jnp.zeros((16,), jnp.float32)
        for d in range(LATENT_DIM):
            cols = (lane + d) & (LATENT_DIM - 1)
            p = plsc.load_gather(pro_v, [rows, cols])
            q = plsc.load_gather(diff_v, [rows, cols])
            accp = accp + _sigmoid(p)
            accd = accd + _sigmoid(q)
        kv = k_v[pl.ds(g * 16, 16)]
        disc = 2.0 * _sigmoid(kv)
        out_v[pl.ds(g * 16, 16)] = _sigmoid((accp - accd) * disc)
        return carry

    lax.fori_loop(0, N_GROUPS, group, 0)

    pltpu.sync_copy(out_v, out_hbm.at[pl.ds(base, B_PER_W)])


@jax.jit
def _mirt(user, item, pro_weight, diff_weight, k_flat):
    mesh = plsc.VectorSubcoreMesh(
        core_axis_name="c", subcore_axis_name="s",
        num_cores=NC, num_subcores=NS)
    return pl.kernel(
        _body,
        out_type=jax.ShapeDtypeStruct((BATCH,), jnp.float32),
        mesh=mesh,
        scratch_types=[
            pltpu.VMEM((B_PER_W,), jnp.int32),
            pltpu.VMEM((B_PER_W,), jnp.int32),
            pltpu.VMEM((B_PER_W, LATENT_DIM), jnp.float32),
            pltpu.VMEM((B_PER_W, LATENT_DIM), jnp.float32),
            pltpu.VMEM((B_PER_W,), jnp.float32),
            pltpu.VMEM((B_PER_W,), jnp.float32),
            pltpu.SemaphoreType.DMA,
            pltpu.SemaphoreType.DMA,
            pltpu.SemaphoreType.DMA,
        ],
        compiler_params=pltpu.CompilerParams(
            needs_layout_passes=False, use_tc_tiling_on_sc=False),
        name="mirtnet_sc",
    )(user, item, pro_weight, diff_weight, k_flat)


def kernel(user, item, pro_weight, diff_weight, exercise_k_weight):
    user = user.astype(jnp.int32)
    item = item.astype(jnp.int32)
    k_flat = exercise_k_weight.reshape(-1)
    return _mirt(user, item, pro_weight, diff_weight, k_flat)


# speedup vs baseline: 1.4581x; 1.4581x over previous
import jax
import jax.numpy as jnp
from jax import lax
from jax.experimental import pallas as pl
from jax.experimental.pallas import tpu as pltpu
from jax.experimental.pallas import tpu_sc as plsc

BATCH = 16384
LATENT_DIM = 32
NC = 2
NS = 16
NW = NC * NS
B_PER_W = BATCH // NW
CHUNK = 128
N_CHUNKS = B_PER_W // CHUNK
GROUPS = CHUNK // 16


def _sigmoid(x):
    return 1.0 / (1.0 + jnp.exp(-x))


def _body(user_hbm, item_hbm, pro_hbm, diff_hbm, k_hbm, out_hbm,
          u_v, it_v, pro_t, diff_t, kv_big, out_v, sem_p, sem_d, sem_k):
    wid = lax.axis_index("s") * NC + lax.axis_index("c")
    base = wid * B_PER_W

    pltpu.sync_copy(user_hbm.at[pl.ds(base, B_PER_W)], u_v)
    pltpu.sync_copy(item_hbm.at[pl.ds(base, B_PER_W)], it_v)

    lane = lax.iota(jnp.int32, 16)
    zeros = jnp.zeros((16,), jnp.int32)

    def chunk_body(c, carry):
        c0 = pl.multiple_of(c * CHUNK, CHUNK)

        def enq(g, inner):
            g16 = pl.multiple_of(g * 16, 16)
            u16 = u_v[pl.ds(c0 + g16, 16)]
            it16 = it_v[pl.ds(c0 + g16, 16)]
            for j in range(16):
                e = g16 + j
                u = u16[j]
                it = it16[j]
                pltpu.make_async_copy(
                    pro_hbm.at[pl.ds(u, 1), :], pro_t.at[pl.ds(e, 1), :],
                    sem_p
                ).start()
                pltpu.make_async_copy(
                    diff_hbm.at[pl.ds(it, 1), :], diff_t.at[pl.ds(e, 1), :],
                    sem_d
                ).start()
                pltpu.make_async_copy(
                    k_hbm.at[pl.ds(it, 1), :],
                    kv_big.at[pl.ds(c0 + e, 1), :], sem_k
                ).start()
            return inner

        lax.fori_loop(0, GROUPS, enq, 0)

        def drain(e, inner):
            pltpu.make_async_copy(
                pro_hbm.at[pl.ds(0, 1), :], pro_t.at[pl.ds(e, 1), :], sem_p
            ).wait()
            pltpu.make_async_copy(
                diff_hbm.at[pl.ds(0, 1), :], diff_t.at[pl.ds(e, 1), :], sem_d
            ).wait()
            pltpu.make_async_copy(
                k_hbm.at[pl.ds(0, 1), :], kv_big.at[pl.ds(0, 1), :], sem_k
            ).wait()
            return inner

        lax.fori_loop(0, CHUNK, drain, 0)

        def comp(g, inner):
            g16 = pl.multiple_of(g * 16, 16)
            acc = jnp.zeros((16,), jnp.float32)
            for j in range(16):
                e = g16 + j
                p0 = pro_t[e, pl.ds(0, 16)]
                p1 = pro_t[e, pl.ds(16, 16)]
                q0 = diff_t[e, pl.ds(0, 16)]
                q1 = diff_t[e, pl.ds(16, 16)]
                s = (_sigmoid(p0) - _sigmoid(q0)) + (
                    _sigmoid(p1) - _sigmoid(q1))
                tot = jnp.sum(s, axis=0)
                acc = jnp.where(lane == j, tot, acc)
            k16 = plsc.load_gather(kv_big, [c0 + g16 + lane, zeros])
            out_v[pl.ds(c0 + g16, 16)] = _sigmoid(
                acc * (2.0 * _sigmoid(k16)))
            return inner

        lax.fori_loop(0, GROUPS, comp, 0)
        return carry

    lax.fori_loop(0, N_CHUNKS, chunk_body, 0)

    pltpu.sync_copy(out_v, out_hbm.at[pl.ds(base, B_PER_W)])


@jax.jit
def _mirt(user, item, pro_weight, diff_weight, exercise_k_weight):
    mesh = plsc.VectorSubcoreMesh(
        core_axis_name="c", subcore_axis_name="s",
        num_cores=NC, num_subcores=NS)
    return pl.kernel(
        _body,
        out_type=jax.ShapeDtypeStruct((BATCH,), jnp.float32),
        mesh=mesh,
        scratch_types=[
            pltpu.VMEM((B_PER_W,), jnp.int32),
            pltpu.VMEM((B_PER_W,), jnp.int32),
            pltpu.VMEM((CHUNK, LATENT_DIM), jnp.float32),
            pltpu.VMEM((CHUNK, LATENT_DIM), jnp.float32),
            pltpu.VMEM((B_PER_W, 1), jnp.float32),
            pltpu.VMEM((B_PER_W,), jnp.float32),
            pltpu.SemaphoreType.DMA,
            pltpu.SemaphoreType.DMA,
            pltpu.SemaphoreType.DMA,
        ],
        compiler_params=pltpu.CompilerParams(
            needs_layout_passes=False, use_tc_tiling_on_sc=True),
        name="mirtnet_sc",
    )(user, item, pro_weight, diff_weight, exercise_k_weight)


def kernel(user, item, pro_weight, diff_weight, exercise_k_weight):
    user = user.astype(jnp.int32)
    item = item.astype(jnp.int32)
    return _mirt(user, item, pro_weight, diff_weight, exercise_k_weight)


# speedup vs baseline: 2.4601x; 1.6872x over previous
import jax
import jax.numpy as jnp
from jax import lax
from jax.experimental import pallas as pl
from jax.experimental.pallas import tpu as pltpu
from jax.experimental.pallas import tpu_sc as plsc

BATCH = 16384
LATENT_DIM = 32
NC = 2
NS = 16
NW = NC * NS
B_PER_W = BATCH // NW
CHUNK = 128
N_CHUNKS = B_PER_W // CHUNK
GROUPS = CHUNK // 16


def _sigmoid(x):
    return 1.0 / (1.0 + jnp.exp(-x))


def _body(user_hbm, item_hbm, pro_hbm, diff_hbm, k_hbm, out_hbm,
          u_v, it_v, pro_t, diff_t, kv_big, out_v, sem_p, sem_d, sem_k):
    wid = lax.axis_index("s") * NC + lax.axis_index("c")
    base = wid * B_PER_W

    pltpu.sync_copy(user_hbm.at[pl.ds(base, B_PER_W)], u_v)
    pltpu.sync_copy(item_hbm.at[pl.ds(base, B_PER_W)], it_v)

    lane = lax.iota(jnp.int32, 16)
    zeros = jnp.zeros((16,), jnp.int32)

    def chunk_body(c, carry):
        c0 = pl.multiple_of(c * CHUNK, CHUNK)

        def enq(g, inner):
            g16 = pl.multiple_of(g * 16, 16)
            u16 = u_v[pl.ds(c0 + g16, 16)]
            it16 = it_v[pl.ds(c0 + g16, 16)]
            for j in range(16):
                e = g16 + j
                u = u16[j]
                it = it16[j]
                pltpu.make_async_copy(
                    pro_hbm.at[u // 8, pl.ds(u % 8, 1), :],
                    pro_t.at[pl.ds(e, 1), :], sem_p
                ).start()
                pltpu.make_async_copy(
                    diff_hbm.at[it // 8, pl.ds(it % 8, 1), :],
                    diff_t.at[pl.ds(e, 1), :], sem_d
                ).start()
                pltpu.make_async_copy(
                    k_hbm.at[it // 8, pl.ds(it % 8, 1), :],
                    kv_big.at[pl.ds(c0 + e, 1), :], sem_k
                ).start()
            return inner

        lax.fori_loop(0, GROUPS, enq, 0)

        def drain(e, inner):
            pltpu.make_async_copy(
                pro_hbm.at[0, pl.ds(0, 1), :],
                pro_t.at[pl.ds(e, 1), :], sem_p
            ).wait()
            pltpu.make_async_copy(
                diff_hbm.at[0, pl.ds(0, 1), :],
                diff_t.at[pl.ds(e, 1), :], sem_d
            ).wait()
            pltpu.make_async_copy(
                k_hbm.at[0, pl.ds(0, 1), :],
                kv_big.at[pl.ds(0, 1), :], sem_k
            ).wait()
            return inner

        lax.fori_loop(0, CHUNK, drain, 0)

        def comp(g, inner):
            g16 = pl.multiple_of(g * 16, 16)
            acc = jnp.zeros((16,), jnp.float32)
            for j in range(16):
                e = g16 + j
                p0 = pro_t[e, pl.ds(0, 16)]
                p1 = pro_t[e, pl.ds(16, 16)]
                q0 = diff_t[e, pl.ds(0, 16)]
                q1 = diff_t[e, pl.ds(16, 16)]
                s = (_sigmoid(p0) - _sigmoid(q0)) + (
                    _sigmoid(p1) - _sigmoid(q1))
                tot = jnp.sum(s, axis=0)
                acc = jnp.where(lane == j, tot, acc)
            k16 = plsc.load_gather(kv_big, [c0 + g16 + lane, zeros])
            out_v[pl.ds(c0 + g16, 16)] = _sigmoid(
                acc * (2.0 * _sigmoid(k16)))
            return inner

        lax.fori_loop(0, GROUPS, comp, 0)
        return carry

    lax.fori_loop(0, N_CHUNKS, chunk_body, 0)

    pltpu.sync_copy(out_v, out_hbm.at[pl.ds(base, B_PER_W)])


@jax.jit
def _mirt(user, item, pro_weight, diff_weight, exercise_k_weight):
    mesh = plsc.VectorSubcoreMesh(
        core_axis_name="c", subcore_axis_name="s",
        num_cores=NC, num_subcores=NS)
    return pl.kernel(
        _body,
        out_type=jax.ShapeDtypeStruct((BATCH,), jnp.float32),
        mesh=mesh,
        scratch_types=[
            pltpu.VMEM((B_PER_W,), jnp.int32),
            pltpu.VMEM((B_PER_W,), jnp.int32),
            pltpu.VMEM((CHUNK, LATENT_DIM), jnp.float32),
            pltpu.VMEM((CHUNK, LATENT_DIM), jnp.float32),
            pltpu.VMEM((B_PER_W, 1), jnp.float32),
            pltpu.VMEM((B_PER_W,), jnp.float32),
            pltpu.SemaphoreType.DMA,
            pltpu.SemaphoreType.DMA,
            pltpu.SemaphoreType.DMA,
        ],
        compiler_params=pltpu.CompilerParams(
            needs_layout_passes=False, use_tc_tiling_on_sc=True),
        name="mirtnet_sc",
    )(user, item, pro_weight, diff_weight, exercise_k_weight)


def kernel(user, item, pro_weight, diff_weight, exercise_k_weight):
    user = user.astype(jnp.int32)
    item = item.astype(jnp.int32)
    pro3 = pro_weight.reshape(-1, 8, LATENT_DIM)
    diff3 = diff_weight.reshape(-1, 8, LATENT_DIM)
    k3 = exercise_k_weight.reshape(-1, 8, 1)
    return _mirt(user, item, pro3, diff3, k3)
